# baseline (speedup 1.0000x reference)
import functools

import jax
import jax.numpy as jnp
from jax import lax
from jax.experimental import pallas as pl
from jax.experimental.pallas import tpu as pltpu
from jax.experimental.pallas import tpu_sc as plsc

_VHI, _VLO = 8, 128
_NPAD = _VHI * _VLO
_K = 600
_CHUNK = 96


def _topk_body(conf_ref, conf_out_ref, idx_out_ref, *, n):
    nc, bs = conf_ref.shape[0], conf_ref.shape[1]
    key = conf_ref[0]
    for c in range(1, nc):
        key = jnp.maximum(key, conf_ref[c])

    shape = (bs, _VHI, _VLO)
    v = (lax.broadcasted_iota(jnp.int32, shape, 1) * _VLO
         + lax.broadcasted_iota(jnp.int32, shape, 2))
    b = lax.broadcasted_iota(jnp.int32, shape, 0)
    payload = b * n + v

    kk = 2
    while kk <= _NPAD:
        s = kk // 2
        while s >= 1:
            if s < _VLO:
                ax, sh = 2, s
            else:
                ax, sh = 1, s // _VLO
            upper = (v & s) != 0
            pk = jnp.where(upper, jnp.roll(key, sh, axis=ax),
                           jnp.roll(key, -sh, axis=ax))
            pi = jnp.where(upper, jnp.roll(payload, sh, axis=ax),
                           jnp.roll(payload, -sh, axis=ax))
            precedes = (key > pk) | ((key == pk) & (payload < pi))
            want_small = jnp.logical_not(
                jnp.logical_xor((v & kk) == 0, jnp.logical_not(upper)))
            take = jnp.logical_not(jnp.logical_xor(precedes, want_small))
            key = jnp.where(take, key, pk)
            payload = jnp.where(take, payload, pi)
            s //= 2
        kk *= 2

    conf_out_ref[...] = 1.0 / (1.0 + jnp.exp(-key))
    idx_out_ref[...] = payload


def _tc_topk(conf_t, n):
    nc, bs = conf_t.shape[0], conf_t.shape[1]
    return pl.pallas_call(
        functools.partial(_topk_body, n=n),
        out_shape=[
            jax.ShapeDtypeStruct((bs, _VHI, _VLO), jnp.float32),
            jax.ShapeDtypeStruct((bs, _VHI, _VLO), jnp.int32),
        ],
    )(conf_t)


def _sc_gather(feat_flat, anc_flat, idx_flat, total_rows, n, d, ad_pad):
    info = plsc.get_sparse_core_info()
    ncores = info.num_cores
    nw = ncores * info.num_subcores
    rows_per_w = total_rows // nw
    nch = rows_per_w // _CHUNK
    assert rows_per_w % _CHUNK == 0
    batches_per_w = (total_rows // _K) // nw
    span_rows = batches_per_w * n
    mesh = plsc.VectorSubcoreMesh(core_axis_name="c", subcore_axis_name="s")

    @functools.partial(
        pl.kernel,
        out_type=[
            jax.ShapeDtypeStruct((total_rows, d), jnp.float32),
            jax.ShapeDtypeStruct((total_rows * ad_pad,), jnp.float32),
        ],
        mesh=mesh,
        scratch_types=[
            pltpu.VMEM((rows_per_w,), jnp.int32),
            pltpu.VMEM((2, _CHUNK, d), jnp.float32),
            pltpu.VMEM((2, _CHUNK * ad_pad), jnp.float32),
            pltpu.VMEM((span_rows * ad_pad,), jnp.float32),
            pltpu.SemaphoreType.DMA,
        ],
    )
    def k(feat_hbm, anc_hbm, idx_hbm, feat_out, anc_out,
          idx_v, fbuf, abuf, anc_span, fsem):
        wid = lax.axis_index("s") * ncores + lax.axis_index("c")
        base = wid * rows_per_w
        span_base = wid * span_rows
        pltpu.sync_copy(idx_hbm.at[pl.ds(base, rows_per_w)], idx_v)
        pltpu.sync_copy(
            anc_hbm.at[pl.ds(span_base * ad_pad, span_rows * ad_pad)], anc_span)
        for g in range(nch):
            slot = g % 2
            fcp = pltpu.async_copy(
                feat_hbm.at[idx_v.at[pl.ds(g * _CHUNK, _CHUNK)]],
                fbuf.at[slot], fsem)

            def group_body(t, _):
                ivec = idx_v[pl.ds(g * _CHUNK + t * 16, 16)] - span_base
                for l in range(16):
                    rj = ivec[l]
                    abuf[slot, pl.ds((t * 16 + l) * ad_pad, ad_pad)] = (
                        anc_span[pl.ds(rj * ad_pad, ad_pad)])
                return ()

            lax.fori_loop(0, _CHUNK // 16, group_body, ())
            fcp.wait()
            pltpu.sync_copy(fbuf.at[slot],
                            feat_out.at[pl.ds(base + g * _CHUNK, _CHUNK)])
            pltpu.sync_copy(
                abuf.at[slot],
                anc_out.at[pl.ds((base + g * _CHUNK) * ad_pad,
                                 _CHUNK * ad_pad)])

    return k(feat_flat, anc_flat, idx_flat)


def kernel(instance_feature, anchor, confidence):
    bs, n, d = instance_feature.shape
    ad = anchor.shape[-1]
    nc = confidence.shape[-1]

    conf_pad = jnp.pad(confidence, ((0, 0), (0, _NPAD - n), (0, 0)),
                       constant_values=-jnp.inf)
    conf_t = conf_pad.transpose(2, 0, 1).reshape(nc, bs, _VHI, _VLO)
    conf_sorted, flat_sorted = _tc_topk(conf_t, n)

    top_conf = conf_sorted.reshape(bs, _NPAD)[:, :_K]
    flat_idx = flat_sorted.reshape(bs, _NPAD)[:, :_K].reshape(-1)

    ad_pad = 16
    feat_flat = instance_feature.reshape(bs * n, d)
    anc_flat = jnp.pad(anchor, ((0, 0), (0, 0), (0, ad_pad - ad))).reshape(
        bs * n * ad_pad)

    feat_out, anc_out = _sc_gather(feat_flat, anc_flat, flat_idx, bs * _K, n,
                                   d, ad_pad)
    return (top_conf,
            feat_out.reshape(bs, _K, d),
            anc_out.reshape(bs, _K, ad_pad)[:, :, :ad])
